# baseline (speedup 1.0000x reference)
import functools

import jax
import jax.numpy as jnp
from jax import lax
from jax.experimental import pallas as pl
from jax.experimental.pallas import tpu as pltpu
from jax.experimental.pallas import tpu_sc as plsc

_DIM = 64
_NC = 2
_NS = 16
_NW = _NC * _NS
_GRP = 16


@functools.lru_cache(maxsize=None)
def _make_gather(b, s, vocab):
    n = b * s
    assert n % _NW == 0 and s % (n // _NW) == 0
    per_w = n // _NW
    assert per_w % _GRP == 0
    n_groups = per_w // _GRP
    mesh = plsc.VectorSubcoreMesh(core_axis_name="c", subcore_axis_name="s")

    @functools.partial(
        pl.kernel,
        out_type=jax.ShapeDtypeStruct((b, s, _DIM, _DIM), jnp.float32),
        mesh=mesh,
        scratch_types=[
            pltpu.VMEM((per_w,), jnp.int32),
            pltpu.SemaphoreType.DMA,
        ],
    )
    def gather(maps_hbm, idx_hbm, out_hbm, idx_v, sem):
        wid = lax.axis_index("s") * _NC + lax.axis_index("c")
        base = wid * per_w
        batch = base // s
        srow = base % s
        pltpu.sync_copy(idx_hbm.at[pl.ds(base, per_w)], idx_v)

        def group_body(g, carry):
            ivec = idx_v[pl.ds(g * _GRP, _GRP)]
            for j in range(_GRP):
                v = ivec[j]
                pltpu.async_copy(
                    maps_hbm.at[v],
                    out_hbm.at[batch, srow + g * _GRP + j],
                    sem,
                )
            for j in range(_GRP):
                pltpu.make_async_copy(
                    maps_hbm.at[0], out_hbm.at[batch, srow], sem
                ).wait()
            return carry

        lax.fori_loop(0, n_groups, group_body, 0)

    return gather


def kernel(position_ids, maps):
    b, s = position_ids.shape
    vocab = maps.shape[0]
    idx = position_ids.reshape(b * s)
    return _make_gather(b, s, vocab)(maps, idx)


# speedup vs baseline: 16.5940x; 16.5940x over previous
import functools

import jax
import jax.numpy as jnp
from jax import lax
from jax.experimental import pallas as pl
from jax.experimental.pallas import tpu as pltpu
from jax.experimental.pallas import tpu_sc as plsc

_DIM = 64
_NC = 2
_NS = 16
_NW = _NC * _NS
_GRP = 16
_CHUNK = 4
_NBUF = 2
_CPG = _GRP // _CHUNK


@functools.lru_cache(maxsize=None)
def _make_gather(b, s, vocab):
    n = b * s
    assert n % _NW == 0 and s % (n // _NW) == 0
    per_w = n // _NW
    assert per_w % _GRP == 0 and _CPG % _NBUF == 0
    n_groups = per_w // _GRP
    n_chunks = per_w // _CHUNK
    mesh = plsc.VectorSubcoreMesh(core_axis_name="c", subcore_axis_name="s")

    @functools.partial(
        pl.kernel,
        out_type=jax.ShapeDtypeStruct((b, s, _DIM, _DIM), jnp.float32),
        mesh=mesh,
        scratch_types=[
            pltpu.VMEM((per_w,), jnp.int32),
            [pltpu.VMEM((_CHUNK, _DIM, _DIM), jnp.float32) for _ in range(_NBUF)],
            [pltpu.SemaphoreType.DMA for _ in range(_NBUF)],
            [pltpu.SemaphoreType.DMA for _ in range(_NBUF)],
        ],
    )
    def gather(maps_hbm, idx_hbm, out_hbm, idx_v, bufs, gsems, osems):
        wid = lax.axis_index("s") * _NC + lax.axis_index("c")
        base = wid * per_w
        batch = base // s
        srow = base % s
        pltpu.sync_copy(idx_hbm.at[pl.ds(base, per_w)], idx_v)

        def start_gather(vals, buf_i):
            for j in range(_CHUNK):
                pltpu.async_copy(
                    maps_hbm.at[vals[j]], bufs[buf_i].at[j], gsems[buf_i]
                )

        def wait_gather(buf_i):
            for j in range(_CHUNK):
                pltpu.make_async_copy(
                    maps_hbm.at[0], bufs[buf_i].at[j], gsems[buf_i]
                ).wait()

        def start_out(c, buf_i):
            pltpu.async_copy(
                bufs[buf_i],
                out_hbm.at[batch, pl.ds(srow + c * _CHUNK, _CHUNK)],
                osems[buf_i],
            )

        def wait_out(buf_i):
            pltpu.make_async_copy(
                bufs[buf_i], out_hbm.at[batch, pl.ds(srow, _CHUNK)], osems[buf_i]
            ).wait()

        def group_body(g, carry):
            ivec = idx_v[pl.ds(g * _GRP, _GRP)]
            for k in range(_CPG):
                c = g * _CPG + k
                buf_i = k % _NBUF
                vals = [ivec[k * _CHUNK + j] for j in range(_CHUNK)]

                if k >= _NBUF:
                    wait_out(buf_i)
                else:
                    @pl.when(g >= 1)
                    def _():
                        wait_out(buf_i)

                start_gather(vals, buf_i)

                prev = (buf_i - 1) % _NBUF
                if k >= 1:
                    wait_gather(prev)
                    start_out(c - 1, prev)
                else:
                    @pl.when(g >= 1)
                    def _():
                        wait_gather(prev)
                        start_out(c - 1, prev)

            return carry

        lax.fori_loop(0, n_groups, group_body, 0)
        last_buf = (n_chunks - 1) % _NBUF
        wait_gather(last_buf)
        start_out(n_chunks - 1, last_buf)
        for buf_i in range(_NBUF):
            wait_out(buf_i)

    return gather


def kernel(position_ids, maps):
    b, s = position_ids.shape
    vocab = maps.shape[0]
    idx = position_ids.reshape(b * s)
    return _make_gather(b, s, vocab)(maps, idx)


# speedup vs baseline: 16.8363x; 1.0146x over previous
import functools

import jax
import jax.numpy as jnp
from jax import lax
from jax.experimental import pallas as pl
from jax.experimental.pallas import tpu as pltpu
from jax.experimental.pallas import tpu_sc as plsc

_DIM = 64
_ROW = _DIM * _DIM
_NC = 2
_NS = 16
_NW = _NC * _NS
_W = 8
_NBUF = 2


@functools.lru_cache(maxsize=None)
def _make_gather(n, vocab):
    assert n % (_NW * _W) == 0
    per_w = n // _NW
    n_win = per_w // _W
    assert n_win % _NBUF == 0
    mesh = plsc.VectorSubcoreMesh(core_axis_name="c", subcore_axis_name="s")

    @functools.partial(
        pl.kernel,
        out_type=jax.ShapeDtypeStruct((n, _ROW), jnp.float32),
        mesh=mesh,
        scratch_types=[
            pltpu.VMEM((n_win, _W), jnp.int32),
            [pltpu.VMEM((_W, _ROW), jnp.float32) for _ in range(_NBUF)],
            [pltpu.SemaphoreType.DMA for _ in range(_NBUF)],
            [pltpu.SemaphoreType.DMA for _ in range(_NBUF)],
        ],
    )
    def gather(maps_hbm, idx_hbm, out_hbm, idx_v, bufs, gsems, osems):
        wid = lax.axis_index("s") * _NC + lax.axis_index("c")
        base = wid * per_w
        pltpu.sync_copy(idx_hbm.at[pl.ds(wid * n_win, n_win)], idx_v)

        def start_gather(w, buf_i):
            pltpu.async_copy(maps_hbm.at[idx_v.at[w]], bufs[buf_i], gsems[buf_i])

        def wait_gather(buf_i):
            pltpu.make_async_copy(
                maps_hbm.at[idx_v.at[0]], bufs[buf_i], gsems[buf_i]
            ).wait()

        def start_out(w, buf_i):
            pltpu.async_copy(
                bufs[buf_i], out_hbm.at[pl.ds(base + w * _W, _W)], osems[buf_i]
            )

        def wait_out(buf_i):
            pltpu.make_async_copy(
                bufs[buf_i], out_hbm.at[pl.ds(base, _W)], osems[buf_i]
            ).wait()

        start_gather(0, 0)

        def body(g, carry):
            for b in range(_NBUF):
                w = g * _NBUF + b
                wait_gather(b)
                start_out(w, b)
                nxt = (b + 1) % _NBUF
                if b + 1 < _NBUF:
                    @pl.when(g >= 1)
                    def _():
                        wait_out(nxt)
                else:
                    wait_out(nxt)
                @pl.when(w + 1 < n_win)
                def _():
                    start_gather(w + 1, nxt)
            return carry

        lax.fori_loop(0, n_win // _NBUF, body, 0)
        wait_out((n_win - 1) % _NBUF)

    return gather


def kernel(position_ids, maps):
    b, s = position_ids.shape
    n = b * s
    vocab = maps.shape[0]
    idx = position_ids.reshape(n // _W, _W)
    maps2 = maps.reshape(vocab, _ROW)
    out = _make_gather(n, vocab)(maps2, idx)
    return out.reshape(b, s, _DIM, _DIM)


# speedup vs baseline: 16.8832x; 1.0028x over previous
import functools

import jax
import jax.numpy as jnp
from jax import lax
from jax.experimental import pallas as pl
from jax.experimental.pallas import tpu as pltpu
from jax.experimental.pallas import tpu_sc as plsc

_DIM = 64
_NC = 2
_NS = 16
_NW = _NC * _NS
_W = 8
_NBUF = 2


@functools.lru_cache(maxsize=None)
def _make_gather(b, s, vocab):
    n = b * s
    assert n % (_NW * _W) == 0 and s % (n // _NW) == 0
    per_w = n // _NW
    n_win = per_w // _W
    assert n_win % _NBUF == 0
    mesh = plsc.VectorSubcoreMesh(core_axis_name="c", subcore_axis_name="s")

    @functools.partial(
        pl.kernel,
        out_type=jax.ShapeDtypeStruct((n, _DIM * _DIM), jnp.float32),
        mesh=mesh,
        scratch_types=[
            pltpu.VMEM((per_w,), jnp.int32),
            [pltpu.VMEM((_W, _DIM * _DIM), jnp.float32) for _ in range(_NBUF)],
            [pltpu.SemaphoreType.DMA for _ in range(_NBUF)],
            [pltpu.SemaphoreType.DMA for _ in range(_NBUF)],
        ],
    )
    def gather(maps_hbm, idx_hbm, out_hbm, idx_v, bufs, gsems, osems):
        wid = lax.axis_index("s") * _NC + lax.axis_index("c")
        base = wid * per_w
        pltpu.sync_copy(idx_hbm.at[pl.ds(base, per_w)], idx_v)

        def start_gather(w, buf_i):
            pltpu.async_copy(
                maps_hbm.at[idx_v.at[pl.ds(w * _W, _W)]], bufs[buf_i], gsems[buf_i]
            )

        def wait_gather(buf_i):
            pltpu.make_async_copy(
                maps_hbm.at[idx_v.at[pl.ds(0, _W)]], bufs[buf_i], gsems[buf_i]
            ).wait()

        def start_out(w, buf_i):
            pltpu.async_copy(
                bufs[buf_i],
                out_hbm.at[pl.ds(base + w * _W, _W)],
                osems[buf_i],
            )

        def wait_out(buf_i):
            pltpu.make_async_copy(
                bufs[buf_i], out_hbm.at[pl.ds(base, _W)], osems[buf_i]
            ).wait()

        start_gather(0, 0)

        def body(g, carry):
            for bi in range(_NBUF):
                w = g * _NBUF + bi
                wait_gather(bi)
                start_out(w, bi)
                nxt = (bi + 1) % _NBUF
                if bi + 1 < _NBUF:
                    @pl.when(g >= 1)
                    def _():
                        wait_out(nxt)
                else:
                    wait_out(nxt)
                @pl.when(w + 1 < n_win)
                def _():
                    start_gather(w + 1, nxt)
            return carry

        lax.fori_loop(0, n_win // _NBUF, body, 0)
        wait_out((n_win - 1) % _NBUF)

    return gather


def kernel(position_ids, maps):
    b, s = position_ids.shape
    vocab = maps.shape[0]
    idx = position_ids.reshape(b * s)
    maps2 = maps.reshape(vocab, _DIM * _DIM)
    out = _make_gather(b, s, vocab)(maps2, idx)
    return out.reshape(b, s, _DIM, _DIM)
